# baseline (speedup 1.0000x reference)
import functools

import jax
import jax.numpy as jnp
from jax.experimental import pallas as pl
from jax.experimental.pallas import tpu as pltpu

_F32 = jnp.float32
_BF16 = jnp.bfloat16


def _dot(a, b):
    return jnp.dot(a, b, preferred_element_type=_F32)


def _copy_all(pairs, sem):
    copies = [pltpu.make_async_copy(src, dst, sem) for src, dst in pairs]
    for cp in copies:
        cp.start()
    for cp in copies:
        cp.wait()


def _k1_body(a_ref, zb_hbm, ztb_hbm, t1_hbm,
             abf_ref, q0_ref, q1_ref, xz_hbm, s_hbm,
             zb_s, ztb_s, t1_s, t1bf_s, xz_s, s_s, sem,
             *, rblk, nblk, cblk):
    i = pl.program_id(0)
    sl = pl.ds(i * rblk, rblk)

    @pl.when(i == 0)
    def _prologue():
        _copy_all(((zb_hbm, zb_s), (ztb_hbm, ztb_s), (t1_hbm, t1_s)), sem)
        t1bf_s[...] = t1_s[...].astype(_BF16)
        xz_s[...] = jnp.zeros_like(xz_s)

    a0b = a_ref[0].astype(_BF16)
    a1b = a_ref[1].astype(_BF16)
    abf_ref[0] = a0b
    abf_ref[1] = a1b
    u0 = _dot(a0b, t1bf_s[...])
    u1 = _dot(a1b, t1bf_s[...])
    t1blk = t1_s[sl, :]
    q0_ref[...] = (t1blk + 2.0 * u0).astype(_BF16)
    q1_ref[...] = (t1blk + 2.0 * u1).astype(_BF16)
    for c in range(rblk // cblk):
        csl = pl.ds(i * rblk + c * cblk, cblk)
        r = _dot(zb_s[...], ztb_s[:, csl])
        e = jnp.exp(jnp.maximum(r, 0.0))
        s = jnp.sum(e, axis=0)
        s_s[0, csl] = s
        ebf = (e * (1.0 / s)[None, :]).astype(_BF16)
        xz_s[...] += _dot(ebf, t1bf_s[csl, :])

    @pl.when(i == nblk - 1)
    def _epilogue():
        _copy_all(((xz_s, xz_hbm), (s_s, s_hbm)), sem)


def _k2_body(abf_ref, q0_hbm, q1_hbm, xz_hbm, t1_hbm, s_hbm,
             zb_hbm, ztb_hbm, w1_hbm, w2_hbm,
             acc_ref,
             u0_s, u1_s, xz_s, xz2_s, t1_s, s_s, zb_s, ztb_s, w1_s, w2_s,
             q0_s, q1_s, t2_s, t2bf_s, sem,
             *, rblk, nblk, cblk):
    q = pl.program_id(0)
    i = pl.program_id(1)
    sl = pl.ds(i * rblk, rblk)

    @pl.when((q == 0) & (i == 0))
    def _prologue():
        _copy_all(((q0_hbm, q0_s), (q1_hbm, q1_s), (xz_hbm, xz_s),
                   (t1_hbm, t1_s), (s_hbm, s_s), (zb_hbm, zb_s),
                   (ztb_hbm, ztb_s), (w1_hbm, w1_s), (w2_hbm, w2_s)), sem)
        acc_ref[...] = jnp.zeros_like(acc_ref)
        xz2_s[...] = jnp.zeros_like(xz2_s)

    def az_accum(j):
        for c in range(rblk // cblk):
            csl = pl.ds(j * rblk + c * cblk, cblk)
            r = _dot(zb_s[...], ztb_s[:, csl])
            e = jnp.exp(jnp.maximum(r, 0.0))
            ebf = (e * (1.0 / s_s[0, csl])[None, :]).astype(_BF16)
            xz2_s[...] += _dot(ebf, t2_s[csl, :].astype(_BF16))

    @pl.when(q == 0)
    def _l1_sweep2():
        s0 = _dot(abf_ref[0], q0_s[...])
        s1 = _dot(abf_ref[1], q1_s[...])
        xs = s0 + s1 + xz_s[sl, :] - t1_s[sl, :]
        t2_s[sl, :] = jnp.maximum(jnp.dot(xs, w1_s[...]), 0.0)

        @pl.when(i > 0)
        def _az_prev():
            az_accum(i - 1)

        @pl.when(i == nblk - 1)
        def _finish():
            t2bf_s[...] = t2_s[...].astype(_BF16)

    @pl.when(q == 1)
    def _l2_sweep1():
        @pl.when(i == 0)
        def _az_last():
            az_accum(nblk - 1)

        u0_s[sl, :] = _dot(abf_ref[0], t2bf_s[...])
        u1_s[sl, :] = _dot(abf_ref[1], t2bf_s[...])

        @pl.when(i == nblk - 1)
        def _finish():
            q0_s[...] = (t2_s[...] + 2.0 * u0_s[...]).astype(_BF16)
            q1_s[...] = (t2_s[...] + 2.0 * u1_s[...]).astype(_BF16)

    @pl.when(q == 2)
    def _l2_sweep2():
        s0 = _dot(abf_ref[0], q0_s[...])
        s1 = _dot(abf_ref[1], q1_s[...])
        xs = s0 + s1 + xz2_s[sl, :] - t2_s[sl, :]
        y = jnp.maximum(jnp.dot(xs, w2_s[...]), 0.0)
        acc_ref[...] += jnp.sum(y, axis=0, keepdims=True)


def _gwnet(A, X, Z, W1, W2, *, rblk1=512, rblk2=512, cblk=128, interpret=False):
    B, d, n = X.shape
    dz = Z.shape[1]
    h1 = W2.shape[1]
    C = B * d
    nblk1 = n // rblk1
    nblk2 = n // rblk2
    T1 = X.reshape(C, n).T
    Zb = Z.astype(_BF16)
    Ztb = Zb.T
    Wbd1 = jnp.kron(jnp.eye(B, dtype=W1.dtype), W1)
    Wbd2 = jnp.kron(jnp.eye(B, dtype=W2.dtype), W2)

    any_spec = pl.BlockSpec(memory_space=pl.ANY)

    Abf, Q0, Q1, Xz, s = pl.pallas_call(
        functools.partial(_k1_body, rblk=rblk1, nblk=nblk1, cblk=cblk),
        grid=(nblk1,),
        in_specs=[
            pl.BlockSpec((2, rblk1, n), lambda i: (0, i, 0)),
            any_spec, any_spec, any_spec,
        ],
        out_specs=[
            pl.BlockSpec((2, rblk1, n), lambda i: (0, i, 0)),
            pl.BlockSpec((rblk1, C), lambda i: (i, 0)),
            pl.BlockSpec((rblk1, C), lambda i: (i, 0)),
            any_spec,
            any_spec,
        ],
        out_shape=[
            jax.ShapeDtypeStruct((2, n, n), _BF16),
            jax.ShapeDtypeStruct((n, C), _BF16),
            jax.ShapeDtypeStruct((n, C), _BF16),
            jax.ShapeDtypeStruct((n, C), _F32),
            jax.ShapeDtypeStruct((1, n), _F32),
        ],
        scratch_shapes=[
            pltpu.VMEM((n, dz), _BF16),
            pltpu.VMEM((dz, n), _BF16),
            pltpu.VMEM((n, C), _F32),
            pltpu.VMEM((n, C), _BF16),
            pltpu.VMEM((n, C), _F32),
            pltpu.VMEM((1, n), _F32),
            pltpu.SemaphoreType.DMA,
        ],
        interpret=interpret,
    )(A, Zb, Ztb, T1)

    acc = pl.pallas_call(
        functools.partial(_k2_body, rblk=rblk2, nblk=nblk2, cblk=2 * cblk),
        grid=(3, nblk2),
        in_specs=[
            pl.BlockSpec((2, rblk2, n), lambda q, i: (0, i, 0)),
            any_spec, any_spec, any_spec, any_spec, any_spec,
            any_spec, any_spec, any_spec, any_spec,
        ],
        out_specs=pl.BlockSpec((1, C), lambda q, i: (0, 0)),
        out_shape=jax.ShapeDtypeStruct((1, C), _F32),
        scratch_shapes=[
            pltpu.VMEM((n, C), _F32),
            pltpu.VMEM((n, C), _F32),
            pltpu.VMEM((n, C), _F32),
            pltpu.VMEM((n, C), _F32),
            pltpu.VMEM((n, C), _F32),
            pltpu.VMEM((1, n), _F32),
            pltpu.VMEM((n, dz), _BF16),
            pltpu.VMEM((dz, n), _BF16),
            pltpu.VMEM((C, C), _F32),
            pltpu.VMEM((C, C), _F32),
            pltpu.VMEM((n, C), _BF16),
            pltpu.VMEM((n, C), _BF16),
            pltpu.VMEM((n, C), _F32),
            pltpu.VMEM((n, C), _BF16),
            pltpu.SemaphoreType.DMA,
        ],
        interpret=interpret,
    )(Abf, Q0, Q1, Xz, T1, s, Zb, Ztb, Wbd1, Wbd2)
    return (acc / n).reshape(B, h1)


def kernel(A, X, Z, W1, W2):
    return _gwnet(A, X, Z, W1, W2)


# speedup vs baseline: 1.0043x; 1.0043x over previous
import functools

import jax
import jax.numpy as jnp
from jax.experimental import pallas as pl
from jax.experimental.pallas import tpu as pltpu

_F32 = jnp.float32
_BF16 = jnp.bfloat16


def _dot(a, b):
    return jnp.dot(a, b, preferred_element_type=_F32)


def _copy_all(pairs, sem):
    copies = [pltpu.make_async_copy(src, dst, sem) for src, dst in pairs]
    for cp in copies:
        cp.start()
    for cp in copies:
        cp.wait()


def _k1_body(a_ref, zb_hbm, ztb_hbm, t1_hbm,
             abf_ref, q0_ref, q1_ref, xz_hbm, s_hbm,
             zb_s, ztb_s, t1_s, t1bf_s, xz_s, s_s, sem,
             *, rblk, nblk, cblk):
    i = pl.program_id(0)
    sl = pl.ds(i * rblk, rblk)

    @pl.when(i == 0)
    def _prologue():
        _copy_all(((zb_hbm, zb_s), (ztb_hbm, ztb_s), (t1_hbm, t1_s)), sem)
        t1bf_s[...] = t1_s[...].astype(_BF16)
        xz_s[...] = jnp.zeros_like(xz_s)

    a0b = a_ref[0].astype(_BF16)
    a1b = a_ref[1].astype(_BF16)
    abf_ref[0] = a0b
    abf_ref[1] = a1b
    u0 = _dot(a0b, t1bf_s[...])
    u1 = _dot(a1b, t1bf_s[...])
    t1blk = t1_s[sl, :]
    q0_ref[...] = (t1blk + 2.0 * u0).astype(_BF16)
    q1_ref[...] = (t1blk + 2.0 * u1).astype(_BF16)
    for c in range(rblk // cblk):
        csl = pl.ds(i * rblk + c * cblk, cblk)
        r = _dot(zb_s[...], ztb_s[:, csl])
        e = jnp.exp(jnp.maximum(r, 0.0))
        s = jnp.sum(e, axis=0)
        s_s[0, csl] = s
        ebf = (e * (1.0 / s)[None, :]).astype(_BF16)
        xz_s[...] += _dot(ebf, t1bf_s[csl, :])

    @pl.when(i == nblk - 1)
    def _epilogue():
        _copy_all(((xz_s, xz_hbm), (s_s, s_hbm)), sem)


def _k2_body(abf_ref, q0_hbm, q1_hbm, xz_hbm, t1_hbm, s_hbm,
             zb_hbm, ztb_hbm, w1_hbm, w2_hbm,
             acc_ref,
             u0_s, u1_s, xz_s, t1_s, s_s, zb_s, ztb_s, w1_s, w2_s,
             q0_s, q1_s, t2_s, t2bf_s, sem,
             *, rblk, nblk, cblk):
    q = pl.program_id(0)
    i = pl.program_id(1)
    ie = jnp.where(q == 1, nblk - 1 - i, i)
    sl = pl.ds(ie * rblk, rblk)

    @pl.when((q == 0) & (i == 0))
    def _prologue():
        _copy_all(((q0_hbm, q0_s), (q1_hbm, q1_s), (xz_hbm, xz_s),
                   (t1_hbm, t1_s), (s_hbm, s_s), (zb_hbm, zb_s),
                   (ztb_hbm, ztb_s), (w1_hbm, w1_s), (w2_hbm, w2_s)), sem)
        acc_ref[...] = jnp.zeros_like(acc_ref)

    @pl.when(q == 0)
    def _l1_sweep2():
        s0 = _dot(abf_ref[0], q0_s[...])
        s1 = _dot(abf_ref[1], q1_s[...])
        xs = s0 + s1 + xz_s[sl, :] - t1_s[sl, :]
        t2_s[sl, :] = jnp.maximum(jnp.dot(xs, w1_s[...]), 0.0)

        @pl.when(i == nblk - 1)
        def _finish():
            t2bf_s[...] = t2_s[...].astype(_BF16)

    @pl.when(q == 1)
    def _l2_sweep1():
        @pl.when(i == 0)
        def _init():
            xz_s[...] = jnp.zeros_like(xz_s)

        u0_s[sl, :] = _dot(abf_ref[0], t2bf_s[...])
        u1_s[sl, :] = _dot(abf_ref[1], t2bf_s[...])
        for c in range(rblk // cblk):
            csl = pl.ds(i * rblk + c * cblk, cblk)
            r = _dot(zb_s[...], ztb_s[:, csl])
            e = jnp.exp(jnp.maximum(r, 0.0))
            ebf = (e * (1.0 / s_s[0, csl])[None, :]).astype(_BF16)
            xz_s[...] += _dot(ebf, t2bf_s[csl, :])

        @pl.when(i == nblk - 1)
        def _finish():
            q0_s[...] = (t2_s[...] + 2.0 * u0_s[...]).astype(_BF16)
            q1_s[...] = (t2_s[...] + 2.0 * u1_s[...]).astype(_BF16)

    @pl.when(q == 2)
    def _l2_sweep2():
        s0 = _dot(abf_ref[0], q0_s[...])
        s1 = _dot(abf_ref[1], q1_s[...])
        xs = s0 + s1 + xz_s[sl, :] - t2_s[sl, :]
        y = jnp.maximum(jnp.dot(xs, w2_s[...]), 0.0)
        acc_ref[...] += jnp.sum(y, axis=0, keepdims=True)


def _gwnet(A, X, Z, W1, W2, *, rblk1=512, rblk2=512, cblk=128, interpret=False):
    B, d, n = X.shape
    dz = Z.shape[1]
    h1 = W2.shape[1]
    C = B * d
    nblk1 = n // rblk1
    nblk2 = n // rblk2
    T1 = X.reshape(C, n).T
    Zb = Z.astype(_BF16)
    Ztb = Zb.T
    Wbd1 = jnp.kron(jnp.eye(B, dtype=W1.dtype), W1)
    Wbd2 = jnp.kron(jnp.eye(B, dtype=W2.dtype), W2)

    any_spec = pl.BlockSpec(memory_space=pl.ANY)

    Abf, Q0, Q1, Xz, s = pl.pallas_call(
        functools.partial(_k1_body, rblk=rblk1, nblk=nblk1, cblk=cblk),
        grid=(nblk1,),
        in_specs=[
            pl.BlockSpec((2, rblk1, n), lambda i: (0, i, 0)),
            any_spec, any_spec, any_spec,
        ],
        out_specs=[
            pl.BlockSpec((2, rblk1, n), lambda i: (0, i, 0)),
            pl.BlockSpec((rblk1, C), lambda i: (i, 0)),
            pl.BlockSpec((rblk1, C), lambda i: (i, 0)),
            any_spec,
            any_spec,
        ],
        out_shape=[
            jax.ShapeDtypeStruct((2, n, n), _BF16),
            jax.ShapeDtypeStruct((n, C), _BF16),
            jax.ShapeDtypeStruct((n, C), _BF16),
            jax.ShapeDtypeStruct((n, C), _F32),
            jax.ShapeDtypeStruct((1, n), _F32),
        ],
        scratch_shapes=[
            pltpu.VMEM((n, dz), _BF16),
            pltpu.VMEM((dz, n), _BF16),
            pltpu.VMEM((n, C), _F32),
            pltpu.VMEM((n, C), _BF16),
            pltpu.VMEM((n, C), _F32),
            pltpu.VMEM((1, n), _F32),
            pltpu.SemaphoreType.DMA,
        ],
        interpret=interpret,
    )(A, Zb, Ztb, T1)

    acc = pl.pallas_call(
        functools.partial(_k2_body, rblk=rblk2, nblk=nblk2, cblk=2 * cblk),
        grid=(3, nblk2),
        in_specs=[
            pl.BlockSpec((2, rblk2, n),
                         lambda q, i: (0, jnp.where(q == 1, n // rblk2 - 1 - i, i), 0)),
            any_spec, any_spec, any_spec, any_spec, any_spec,
            any_spec, any_spec, any_spec, any_spec,
        ],
        out_specs=pl.BlockSpec((1, C), lambda q, i: (0, 0)),
        out_shape=jax.ShapeDtypeStruct((1, C), _F32),
        scratch_shapes=[
            pltpu.VMEM((n, C), _F32),
            pltpu.VMEM((n, C), _F32),
            pltpu.VMEM((n, C), _F32),
            pltpu.VMEM((n, C), _F32),
            pltpu.VMEM((1, n), _F32),
            pltpu.VMEM((n, dz), _BF16),
            pltpu.VMEM((dz, n), _BF16),
            pltpu.VMEM((C, C), _F32),
            pltpu.VMEM((C, C), _F32),
            pltpu.VMEM((n, C), _BF16),
            pltpu.VMEM((n, C), _BF16),
            pltpu.VMEM((n, C), _F32),
            pltpu.VMEM((n, C), _BF16),
            pltpu.SemaphoreType.DMA,
        ],
        interpret=interpret,
    )(Abf, Q0, Q1, Xz, T1, s, Zb, Ztb, Wbd1, Wbd2)
    return (acc / n).reshape(B, h1)


def kernel(A, X, Z, W1, W2):
    return _gwnet(A, X, Z, W1, W2)


# speedup vs baseline: 1.0134x; 1.0090x over previous
import functools

import jax
import jax.numpy as jnp
from jax.experimental import pallas as pl
from jax.experimental.pallas import tpu as pltpu

_F32 = jnp.float32
_BF16 = jnp.bfloat16


def _dot(a, b):
    return jnp.dot(a, b, preferred_element_type=_F32)


def _copy_all(pairs, sem):
    copies = [pltpu.make_async_copy(src, dst, sem) for src, dst in pairs]
    for cp in copies:
        cp.start()
    for cp in copies:
        cp.wait()


def _k1_body(a_ref, zb_hbm, ztb_hbm, t1_hbm,
             abf_ref, q0_ref, q1_ref, xz_hbm, s_hbm,
             zb_s, ztb_s, t1_s, t1bf_s, xz_s, s_s, sem,
             *, rblk, nblk, cblk):
    i = pl.program_id(0)
    sl = pl.ds(i * rblk, rblk)

    @pl.when(i == 0)
    def _prologue():
        _copy_all(((zb_hbm, zb_s), (ztb_hbm, ztb_s), (t1_hbm, t1_s)), sem)
        t1bf_s[...] = t1_s[...].astype(_BF16)
        xz_s[...] = jnp.zeros_like(xz_s)

    a0b = a_ref[0].astype(_BF16)
    a1b = a_ref[1].astype(_BF16)
    abf_ref[0] = a0b
    abf_ref[1] = a1b
    u0 = _dot(a0b, t1bf_s[...])
    u1 = _dot(a1b, t1bf_s[...])
    t1blk = t1_s[sl, :]
    q0_ref[...] = (t1blk + 2.0 * u0).astype(_BF16)
    q1_ref[...] = (t1blk + 2.0 * u1).astype(_BF16)
    for c in range(rblk // cblk):
        csl = pl.ds(i * rblk + c * cblk, cblk)
        r = _dot(zb_s[...], ztb_s[:, csl])
        e = jnp.exp(jnp.maximum(r, 0.0))
        s = jnp.sum(e, axis=0)
        s_s[0, csl] = s
        ebf = (e * (1.0 / s)[None, :]).astype(_BF16)
        xz_s[...] += _dot(ebf, t1bf_s[csl, :])

    @pl.when(i == nblk - 1)
    def _epilogue():
        _copy_all(((xz_s, xz_hbm), (s_s, s_hbm)), sem)


def _k2_body(abf_ref, q0_hbm, q1_hbm, xz_hbm, t1_hbm, s_hbm,
             zb_hbm, ztb_hbm, w1_hbm, w2_hbm,
             acc_ref,
             u0_s, u1_s, xz_s, t1_s, s_s, zb_s, ztb_s, w1_s, w2_s,
             q0_s, q1_s, t2_s, t2bf_s, sem,
             *, rblk, nblk, cblk):
    q = pl.program_id(0)
    i = pl.program_id(1)
    ie = jnp.where(q == 1, nblk - 1 - i, i)
    sl = pl.ds(ie * rblk, rblk)

    @pl.when((q == 0) & (i == 0))
    def _prologue():
        _copy_all(((q0_hbm, q0_s), (q1_hbm, q1_s), (xz_hbm, xz_s),
                   (t1_hbm, t1_s), (s_hbm, s_s), (zb_hbm, zb_s),
                   (ztb_hbm, ztb_s), (w1_hbm, w1_s), (w2_hbm, w2_s)), sem)
        acc_ref[...] = jnp.zeros_like(acc_ref)

    @pl.when(q == 0)
    def _l1_sweep2():
        s0 = _dot(abf_ref[0], q0_s[...])
        s1 = _dot(abf_ref[1], q1_s[...])
        xs = s0 + s1 + xz_s[sl, :] - t1_s[sl, :]
        t2_s[sl, :] = jnp.maximum(jnp.dot(xs, w1_s[...]), 0.0)

        @pl.when(i == nblk - 1)
        def _finish():
            t2bf_s[...] = t2_s[...].astype(_BF16)

    @pl.when(q == 1)
    def _l2_sweep1():
        @pl.when(i == 0)
        def _init():
            xz_s[...] = jnp.zeros_like(xz_s)

        u0_s[sl, :] = _dot(abf_ref[0], t2bf_s[...])
        u1_s[sl, :] = _dot(abf_ref[1], t2bf_s[...])
        for c in range(rblk // cblk):
            csl = pl.ds(i * rblk + c * cblk, cblk)
            rb = _dot(zb_s[...], ztb_s[:, csl]).astype(_BF16)
            eb = jnp.exp(jnp.maximum(rb, jnp.bfloat16(0.0)))
            ebf = eb * (1.0 / s_s[0, csl]).astype(_BF16)[None, :]
            xz_s[...] += _dot(ebf, t2bf_s[csl, :])

        @pl.when(i == nblk - 1)
        def _finish():
            q0_s[...] = (t2_s[...] + 2.0 * u0_s[...]).astype(_BF16)
            q1_s[...] = (t2_s[...] + 2.0 * u1_s[...]).astype(_BF16)

    @pl.when(q == 2)
    def _l2_sweep2():
        s0 = _dot(abf_ref[0], q0_s[...])
        s1 = _dot(abf_ref[1], q1_s[...])
        xs = s0 + s1 + xz_s[sl, :] - t2_s[sl, :]
        y = jnp.maximum(jnp.dot(xs, w2_s[...]), 0.0)
        acc_ref[...] += jnp.sum(y, axis=0, keepdims=True)


def _gwnet(A, X, Z, W1, W2, *, rblk1=512, rblk2=512, cblk=128, interpret=False):
    B, d, n = X.shape
    dz = Z.shape[1]
    h1 = W2.shape[1]
    C = B * d
    nblk1 = n // rblk1
    nblk2 = n // rblk2
    T1 = X.reshape(C, n).T
    Zb = Z.astype(_BF16)
    Ztb = Zb.T
    Wbd1 = jnp.kron(jnp.eye(B, dtype=W1.dtype), W1)
    Wbd2 = jnp.kron(jnp.eye(B, dtype=W2.dtype), W2)

    any_spec = pl.BlockSpec(memory_space=pl.ANY)

    Abf, Q0, Q1, Xz, s = pl.pallas_call(
        functools.partial(_k1_body, rblk=rblk1, nblk=nblk1, cblk=cblk),
        grid=(nblk1,),
        in_specs=[
            pl.BlockSpec((2, rblk1, n), lambda i: (0, i, 0)),
            any_spec, any_spec, any_spec,
        ],
        out_specs=[
            pl.BlockSpec((2, rblk1, n), lambda i: (0, i, 0)),
            pl.BlockSpec((rblk1, C), lambda i: (i, 0)),
            pl.BlockSpec((rblk1, C), lambda i: (i, 0)),
            any_spec,
            any_spec,
        ],
        out_shape=[
            jax.ShapeDtypeStruct((2, n, n), _BF16),
            jax.ShapeDtypeStruct((n, C), _BF16),
            jax.ShapeDtypeStruct((n, C), _BF16),
            jax.ShapeDtypeStruct((n, C), _F32),
            jax.ShapeDtypeStruct((1, n), _F32),
        ],
        scratch_shapes=[
            pltpu.VMEM((n, dz), _BF16),
            pltpu.VMEM((dz, n), _BF16),
            pltpu.VMEM((n, C), _F32),
            pltpu.VMEM((n, C), _BF16),
            pltpu.VMEM((n, C), _F32),
            pltpu.VMEM((1, n), _F32),
            pltpu.SemaphoreType.DMA,
        ],
        interpret=interpret,
    )(A, Zb, Ztb, T1)

    acc = pl.pallas_call(
        functools.partial(_k2_body, rblk=rblk2, nblk=nblk2, cblk=rblk2),
        grid=(3, nblk2),
        in_specs=[
            pl.BlockSpec((2, rblk2, n),
                         lambda q, i: (0, jnp.where(q == 1, n // rblk2 - 1 - i, i), 0)),
            any_spec, any_spec, any_spec, any_spec, any_spec,
            any_spec, any_spec, any_spec, any_spec,
        ],
        out_specs=pl.BlockSpec((1, C), lambda q, i: (0, 0)),
        out_shape=jax.ShapeDtypeStruct((1, C), _F32),
        scratch_shapes=[
            pltpu.VMEM((n, C), _F32),
            pltpu.VMEM((n, C), _F32),
            pltpu.VMEM((n, C), _F32),
            pltpu.VMEM((n, C), _F32),
            pltpu.VMEM((1, n), _F32),
            pltpu.VMEM((n, dz), _BF16),
            pltpu.VMEM((dz, n), _BF16),
            pltpu.VMEM((C, C), _F32),
            pltpu.VMEM((C, C), _F32),
            pltpu.VMEM((n, C), _BF16),
            pltpu.VMEM((n, C), _BF16),
            pltpu.VMEM((n, C), _F32),
            pltpu.VMEM((n, C), _BF16),
            pltpu.SemaphoreType.DMA,
        ],
        interpret=interpret,
    )(Abf, Q0, Q1, Xz, T1, s, Zb, Ztb, Wbd1, Wbd2)
    return (acc / n).reshape(B, h1)


def kernel(A, X, Z, W1, W2):
    return _gwnet(A, X, Z, W1, W2)


# speedup vs baseline: 1.0135x; 1.0002x over previous
import functools

import jax
import jax.numpy as jnp
from jax.experimental import pallas as pl
from jax.experimental.pallas import tpu as pltpu

_F32 = jnp.float32
_BF16 = jnp.bfloat16


def _dot(a, b):
    return jnp.dot(a, b, preferred_element_type=_F32)


def _copy_all(pairs, sem):
    copies = [pltpu.make_async_copy(src, dst, sem) for src, dst in pairs]
    for cp in copies:
        cp.start()
    for cp in copies:
        cp.wait()


def _k1_body(a_ref, zb_hbm, ztb_hbm, t1_hbm,
             abf_ref, q0_ref, q1_ref, xz_hbm, s_hbm,
             zb_s, ztb_s, t1_s, t1bf_s, xz_s, s_s, sem,
             *, rblk, nblk, cblk):
    i = pl.program_id(0)
    sl = pl.ds(i * rblk, rblk)

    @pl.when(i == 0)
    def _prologue():
        _copy_all(((zb_hbm, zb_s), (ztb_hbm, ztb_s), (t1_hbm, t1_s)), sem)
        t1bf_s[...] = t1_s[...].astype(_BF16)
        xz_s[...] = jnp.zeros_like(xz_s)

    a0b = a_ref[0].astype(_BF16)
    a1b = a_ref[1].astype(_BF16)
    abf_ref[0] = a0b
    abf_ref[1] = a1b
    u0 = _dot(a0b, t1bf_s[...])
    u1 = _dot(a1b, t1bf_s[...])
    t1blk = t1_s[sl, :]
    q0_ref[...] = (t1blk + 2.0 * u0).astype(_BF16)
    q1_ref[...] = (t1blk + 2.0 * u1).astype(_BF16)
    for c in range(rblk // cblk):
        csl = pl.ds(i * rblk + c * cblk, cblk)
        r = _dot(zb_s[...], ztb_s[:, csl])
        e = jnp.exp(jnp.maximum(r, 0.0))
        s = jnp.sum(e, axis=0)
        s_s[0, csl] = s
        ebf = (e * (1.0 / s)[None, :]).astype(_BF16)
        xz_s[...] += _dot(ebf, t1bf_s[csl, :])

    @pl.when(i == nblk - 1)
    def _epilogue():
        _copy_all(((xz_s, xz_hbm), (s_s, s_hbm)), sem)


def _k2_body(abf_ref, q0_hbm, q1_hbm, xz_hbm, t1_hbm, s_hbm,
             zb_hbm, ztb_hbm, w1_hbm, w2_hbm,
             acc_ref,
             u0_s, u1_s, xz_s, t1_s, s_s, zb_s, ztb_s, w1_s, w2_s,
             q0_s, q1_s, t2_s, t2bf_s, sem,
             *, rblk, nblk, cblk):
    q = pl.program_id(0)
    i = pl.program_id(1)
    ie = jnp.where(q == 1, nblk - 1 - i, i)
    sl = pl.ds(ie * rblk, rblk)

    @pl.when((q == 0) & (i == 0))
    def _prologue():
        _copy_all(((q0_hbm, q0_s), (q1_hbm, q1_s), (xz_hbm, xz_s),
                   (t1_hbm, t1_s), (s_hbm, s_s), (zb_hbm, zb_s),
                   (ztb_hbm, ztb_s), (w1_hbm, w1_s), (w2_hbm, w2_s)), sem)
        acc_ref[...] = jnp.zeros_like(acc_ref)

    @pl.when(q == 0)
    def _l1_sweep2():
        s0 = _dot(abf_ref[0], q0_s[...])
        s1 = _dot(abf_ref[1], q1_s[...])
        xs = s0 + s1 + xz_s[sl, :] - t1_s[sl, :]
        t2_s[sl, :] = jnp.maximum(_dot(xs.astype(_BF16), w1_s[...]), 0.0)

        @pl.when(i == nblk - 1)
        def _finish():
            t2bf_s[...] = t2_s[...].astype(_BF16)

    @pl.when(q == 1)
    def _l2_sweep1():
        @pl.when(i == 0)
        def _init():
            xz_s[...] = jnp.zeros_like(xz_s)

        u0_s[sl, :] = _dot(abf_ref[0], t2bf_s[...])
        u1_s[sl, :] = _dot(abf_ref[1], t2bf_s[...])
        for c in range(rblk // cblk):
            csl = pl.ds(i * rblk + c * cblk, cblk)
            rb = _dot(zb_s[...], ztb_s[:, csl]).astype(_BF16)
            eb = jnp.exp(jnp.maximum(rb, jnp.bfloat16(0.0)))
            ebf = eb * (1.0 / s_s[0, csl]).astype(_BF16)[None, :]
            xz_s[...] += _dot(ebf, t2bf_s[csl, :])

        @pl.when(i == nblk - 1)
        def _finish():
            q0_s[...] = (t2_s[...] + 2.0 * u0_s[...]).astype(_BF16)
            q1_s[...] = (t2_s[...] + 2.0 * u1_s[...]).astype(_BF16)

    @pl.when(q == 2)
    def _l2_sweep2():
        s0 = _dot(abf_ref[0], q0_s[...])
        s1 = _dot(abf_ref[1], q1_s[...])
        xs = s0 + s1 + xz_s[sl, :] - t2_s[sl, :]
        y = jnp.maximum(_dot(xs.astype(_BF16), w2_s[...]), 0.0)
        acc_ref[...] += jnp.sum(y, axis=0, keepdims=True)


def _gwnet(A, X, Z, W1, W2, *, rblk1=512, rblk2=512, cblk=128, interpret=False):
    B, d, n = X.shape
    dz = Z.shape[1]
    h1 = W2.shape[1]
    C = B * d
    nblk1 = n // rblk1
    nblk2 = n // rblk2
    T1 = X.reshape(C, n).T
    Zb = Z.astype(_BF16)
    Ztb = Zb.T
    Wbd1 = jnp.kron(jnp.eye(B, dtype=W1.dtype), W1).astype(_BF16)
    Wbd2 = jnp.kron(jnp.eye(B, dtype=W2.dtype), W2).astype(_BF16)

    any_spec = pl.BlockSpec(memory_space=pl.ANY)

    Abf, Q0, Q1, Xz, s = pl.pallas_call(
        functools.partial(_k1_body, rblk=rblk1, nblk=nblk1, cblk=cblk),
        grid=(nblk1,),
        in_specs=[
            pl.BlockSpec((2, rblk1, n), lambda i: (0, i, 0)),
            any_spec, any_spec, any_spec,
        ],
        out_specs=[
            pl.BlockSpec((2, rblk1, n), lambda i: (0, i, 0)),
            pl.BlockSpec((rblk1, C), lambda i: (i, 0)),
            pl.BlockSpec((rblk1, C), lambda i: (i, 0)),
            any_spec,
            any_spec,
        ],
        out_shape=[
            jax.ShapeDtypeStruct((2, n, n), _BF16),
            jax.ShapeDtypeStruct((n, C), _BF16),
            jax.ShapeDtypeStruct((n, C), _BF16),
            jax.ShapeDtypeStruct((n, C), _F32),
            jax.ShapeDtypeStruct((1, n), _F32),
        ],
        scratch_shapes=[
            pltpu.VMEM((n, dz), _BF16),
            pltpu.VMEM((dz, n), _BF16),
            pltpu.VMEM((n, C), _F32),
            pltpu.VMEM((n, C), _BF16),
            pltpu.VMEM((n, C), _F32),
            pltpu.VMEM((1, n), _F32),
            pltpu.SemaphoreType.DMA,
        ],
        interpret=interpret,
    )(A, Zb, Ztb, T1)

    acc = pl.pallas_call(
        functools.partial(_k2_body, rblk=rblk2, nblk=nblk2, cblk=rblk2),
        grid=(3, nblk2),
        in_specs=[
            pl.BlockSpec((2, rblk2, n),
                         lambda q, i: (0, jnp.where(q == 1, n // rblk2 - 1 - i, i), 0)),
            any_spec, any_spec, any_spec, any_spec, any_spec,
            any_spec, any_spec, any_spec, any_spec,
        ],
        out_specs=pl.BlockSpec((1, C), lambda q, i: (0, 0)),
        out_shape=jax.ShapeDtypeStruct((1, C), _F32),
        scratch_shapes=[
            pltpu.VMEM((n, C), _F32),
            pltpu.VMEM((n, C), _F32),
            pltpu.VMEM((n, C), _F32),
            pltpu.VMEM((n, C), _F32),
            pltpu.VMEM((1, n), _F32),
            pltpu.VMEM((n, dz), _BF16),
            pltpu.VMEM((dz, n), _BF16),
            pltpu.VMEM((C, C), _BF16),
            pltpu.VMEM((C, C), _BF16),
            pltpu.VMEM((n, C), _BF16),
            pltpu.VMEM((n, C), _BF16),
            pltpu.VMEM((n, C), _F32),
            pltpu.VMEM((n, C), _BF16),
            pltpu.SemaphoreType.DMA,
        ],
        interpret=interpret,
    )(Abf, Q0, Q1, Xz, T1, s, Zb, Ztb, Wbd1, Wbd2)
    return (acc / n).reshape(B, h1)


def kernel(A, X, Z, W1, W2):
    return _gwnet(A, X, Z, W1, W2)
